# baseline (speedup 1.0000x reference)
import functools

import jax
import jax.numpy as jnp
from jax import lax
from jax.experimental import pallas as pl
from jax.experimental.pallas import tpu as pltpu
from jax.experimental.pallas import tpu_sc as plsc

_NUM_CORES = 2
_NUM_SUBCORES = 16
_NW = _NUM_CORES * _NUM_SUBCORES
_NBUF = 8


def kernel(indices, table):
    B, L = indices.shape
    V, D = table.shape
    per_w = B // _NW
    n_outer = per_w // _NBUF

    idx = indices.astype(jnp.int32)

    mesh = plsc.VectorSubcoreMesh(core_axis_name="c", subcore_axis_name="s")

    @functools.partial(
        pl.kernel,
        out_type=jax.ShapeDtypeStruct((B, L, D), jnp.float32),
        mesh=mesh,
        compiler_params=pltpu.CompilerParams(use_tc_tiling_on_sc=False),
        scratch_types=[
            pltpu.VMEM((per_w, L), jnp.int32),
            pltpu.VMEM((_NBUF, L, D), jnp.float32),
            [pltpu.SemaphoreType.DMA] * _NBUF,
            [pltpu.SemaphoreType.DMA] * _NBUF,
        ],
    )
    def gather_kernel(idx_hbm, tab_hbm, out_hbm, idx_v, rows_v, gsems, wsems):
        wid = lax.axis_index("s") * _NUM_CORES + lax.axis_index("c")
        base = wid * per_w

        pltpu.sync_copy(idx_hbm.at[pl.ds(base, per_w)], idx_v)

        def start_gather(i, b):
            pltpu.async_copy(tab_hbm.at[idx_v.at[i]], rows_v.at[b], gsems[b])

        def wait_gather(i, b):
            pltpu.make_async_copy(
                tab_hbm.at[idx_v.at[i]], rows_v.at[b], gsems[b]).wait()

        def start_write(i, b):
            pltpu.async_copy(rows_v.at[b], out_hbm.at[base + i], wsems[b])

        def wait_write(i, b):
            pltpu.make_async_copy(
                rows_v.at[b], out_hbm.at[base + i], wsems[b]).wait()

        for b in range(_NBUF):
            start_gather(b, b)

        def outer(g, carry):
            i0 = g * _NBUF
            for b in range(_NBUF):
                wait_gather(i0 + b, b)
                start_write(i0 + b, b)
            for b in range(_NBUF):
                wait_write(i0 + b, b)
                start_gather(i0 + _NBUF + b, b)
            return carry

        lax.fori_loop(0, n_outer - 1, outer, 0)

        i0 = (n_outer - 1) * _NBUF
        for b in range(_NBUF):
            wait_gather(i0 + b, b)
            start_write(i0 + b, b)
        for b in range(_NBUF):
            wait_write(i0 + b, b)

    return gather_kernel(idx, table)
